# Initial kernel scaffold; baseline (speedup 1.0000x reference)
import functools

import jax
import jax.numpy as jnp
from jax import lax
from jax.experimental import pallas as pl
from jax.experimental.pallas import tpu as pltpu
from jax.experimental.pallas import tpu_sc as plsc

N = 10000
D = 128
E = 320000

NCORES = 2
NSUB = 16
NW = NCORES * NSUB

NPAD = 10240
ROWS_PER_TILE = NPAD // NSUB
EPW = E // NW
EB = 128
NB = -(-EPW // EB)
EPW_PAD = NB * EB
DEGW = 16
BN = 512
GRID = NPAD // BN

_f32 = jnp.float32


def _project_body(h_ref, w_ref, out_ref):
    out_ref[...] = lax.dot_general(
        h_ref[...], w_ref[...], (((0,), (0,)), ((), ())),
        preferred_element_type=_f32)


def _project(h_pad, w):
    return pl.pallas_call(
        _project_body,
        grid=(GRID,),
        in_specs=[
            pl.BlockSpec((D, BN), lambda i: (0, i)),
            pl.BlockSpec((D, D), lambda i: (0, 0)),
        ],
        out_specs=pl.BlockSpec((BN, D), lambda i: (i, 0)),
        out_shape=jax.ShapeDtypeStruct((NPAD, D), _f32),
    )(h_pad, w)


def _deg_inv_block(deg_ref):
    deg = deg_ref[0, :, 0] + deg_ref[1, :, 0]
    return 1.0 / jnp.maximum(deg, 1.0)


def _combine1_body(h_ref, agg_ref, deg_ref, w_self_ref, b_ref, w_next_ref,
                   x1_ref, y2_ref):
    xb = lax.dot_general(h_ref[...], w_self_ref[...], (((0,), (0,)), ((), ())),
                         preferred_element_type=_f32)
    aggb = agg_ref[0] + agg_ref[1]
    dinv = _deg_inv_block(deg_ref)
    x1 = jnp.maximum(xb + aggb * dinv[:, None] + b_ref[0][None, :], 0.0)
    x1_ref[...] = x1
    y2_ref[...] = jnp.dot(x1, w_next_ref[...], preferred_element_type=_f32)


def _combine1(h_pad, agg, deg, w_self, b, w_next):
    return pl.pallas_call(
        _combine1_body,
        grid=(GRID,),
        in_specs=[
            pl.BlockSpec((D, BN), lambda i: (0, i)),
            pl.BlockSpec((NCORES, BN, D), lambda i: (0, i, 0)),
            pl.BlockSpec((NCORES, BN, DEGW), lambda i: (0, i, 0)),
            pl.BlockSpec((D, D), lambda i: (0, 0)),
            pl.BlockSpec((1, D), lambda i: (0, 0)),
            pl.BlockSpec((D, D), lambda i: (0, 0)),
        ],
        out_specs=[
            pl.BlockSpec((BN, D), lambda i: (i, 0)),
            pl.BlockSpec((BN, D), lambda i: (i, 0)),
        ],
        out_shape=[
            jax.ShapeDtypeStruct((NPAD, D), _f32),
            jax.ShapeDtypeStruct((NPAD, D), _f32),
        ],
    )(h_pad, agg, deg, w_self, b, w_next)


def _combine2_body(x1_ref, agg_ref, deg_ref, w_self_ref, b_ref, x2_ref):
    xb = jnp.dot(x1_ref[...], w_self_ref[...], preferred_element_type=_f32)
    aggb = agg_ref[0] + agg_ref[1]
    dinv = _deg_inv_block(deg_ref)
    x2_ref[...] = xb + aggb * dinv[:, None] + b_ref[0][None, :]


def _combine2(x1, agg, deg, w_self, b):
    return pl.pallas_call(
        _combine2_body,
        grid=(GRID,),
        in_specs=[
            pl.BlockSpec((BN, D), lambda i: (i, 0)),
            pl.BlockSpec((NCORES, BN, D), lambda i: (0, i, 0)),
            pl.BlockSpec((NCORES, BN, DEGW), lambda i: (0, i, 0)),
            pl.BlockSpec((D, D), lambda i: (0, 0)),
            pl.BlockSpec((1, D), lambda i: (0, 0)),
        ],
        out_specs=pl.BlockSpec((BN, D), lambda i: (i, 0)),
        out_shape=jax.ShapeDtypeStruct((NPAD, D), _f32),
    )(x1, agg, deg, w_self, b)


def _make_sc_scatter(with_deg):
    mesh = plsc.VectorSubcoreMesh(core_axis_name="c", subcore_axis_name="s")

    out_type = [jax.ShapeDtypeStruct((NCORES, NPAD, D), _f32)]
    if with_deg:
        out_type.append(jax.ShapeDtypeStruct((NCORES, NPAD, DEGW), _f32))

    scratch = [
        pltpu.VMEM_SHARED((NPAD, D), _f32),
        pltpu.VMEM((NB, EB), jnp.int32),
        pltpu.VMEM((NB, EB), jnp.int32),
        pltpu.VMEM((EB, D), _f32),
    ]
    if with_deg:
        scratch += [
            pltpu.VMEM_SHARED((NPAD, DEGW), _f32),
            pltpu.VMEM((EB, DEGW), _f32),
        ]

    def body(*refs):
        if with_deg:
            (y_hbm, src_hbm, dst_hbm, agg_hbm, deg_hbm,
             acc_sh, src_v, dst_v, rows_v, deg_sh, ones_v) = refs
        else:
            (y_hbm, src_hbm, dst_hbm, agg_hbm,
             acc_sh, src_v, dst_v, rows_v) = refs

        c = lax.axis_index("c")
        s = lax.axis_index("s")
        w = c * NSUB + s
        base = s * ROWS_PER_TILE

        zero16 = jnp.zeros((16,), _f32)

        def zrow(r, _):
            for j in range(D // 16):
                rows_v[r, pl.ds(j * 16, 16)] = zero16
            return 0

        lax.fori_loop(0, EB, zrow, 0)
        for j in range(ROWS_PER_TILE // EB):
            pltpu.sync_copy(rows_v, acc_sh.at[pl.ds(base + j * EB, EB)])

        if with_deg:
            onehot = jnp.where(lax.iota(jnp.int32, (16,)) == 0, 1.0, 0.0)

            def orow(r, _):
                ones_v[r, pl.ds(0, DEGW)] = onehot
                return 0

            lax.fori_loop(0, EB, orow, 0)
            for j in range(ROWS_PER_TILE // EB):
                pltpu.sync_copy(rows_v.at[:, pl.ds(0, DEGW)],
                                deg_sh.at[pl.ds(base + j * EB, EB)])

        pltpu.sync_copy(src_hbm.at[w], src_v)
        pltpu.sync_copy(dst_hbm.at[w], dst_v)

        plsc.subcore_barrier()

        def batch(i, _):
            pltpu.sync_copy(y_hbm.at[src_v.at[i]], rows_v)
            pltpu.sync_copy(rows_v, acc_sh.at[dst_v.at[i]], add=True)
            if with_deg:
                pltpu.sync_copy(ones_v, deg_sh.at[dst_v.at[i]], add=True)
            return 0

        lax.fori_loop(0, NB, batch, 0)

        plsc.subcore_barrier()

        pltpu.sync_copy(acc_sh.at[pl.ds(base, ROWS_PER_TILE)],
                        agg_hbm.at[c, pl.ds(base, ROWS_PER_TILE)])
        if with_deg:
            pltpu.sync_copy(deg_sh.at[pl.ds(base, ROWS_PER_TILE)],
                            deg_hbm.at[c, pl.ds(base, ROWS_PER_TILE)])

    return pl.kernel(body, out_type=out_type, mesh=mesh,
                     scratch_types=scratch)


_sc_scatter_deg = _make_sc_scatter(with_deg=True)
_sc_scatter = _make_sc_scatter(with_deg=False)


@jax.jit
def kernel(h, edge_index, W1_self, W1_neigh, b1, W2_self, W2_neigh, b2):
    h_pad = jnp.pad(h, ((0, 0), (0, NPAD - N)))

    src = edge_index[0].reshape(NW, EPW)
    dst = edge_index[1].reshape(NW, EPW)
    padlen = EPW_PAD - EPW
    pad_idx = (N + (jnp.arange(padlen, dtype=jnp.int32) % 8))[None, :]
    pad_blk = jnp.broadcast_to(pad_idx, (NW, padlen))
    src_p = jnp.concatenate([src, pad_blk], axis=1).reshape(NW, NB, EB)
    dst_p = jnp.concatenate([dst, pad_blk], axis=1).reshape(NW, NB, EB)

    b1r = b1[None, :]
    b2r = b2[None, :]

    y1 = _project(h_pad, W1_neigh)
    agg1, deg = _sc_scatter_deg(y1, src_p, dst_p)
    x1, y2 = _combine1(h_pad, agg1, deg, W1_self, b1r, W2_neigh)
    (agg2,) = _sc_scatter(y2, src_p, dst_p)
    x2 = _combine2(x1, agg2, deg, W2_self, b2r)
    return x2[:N].T


# speedup vs baseline: 5.3998x; 5.3998x over previous
import functools

import jax
import jax.numpy as jnp
from jax import lax
from jax.experimental import pallas as pl
from jax.experimental.pallas import tpu as pltpu
from jax.experimental.pallas import tpu_sc as plsc

N = 10000
D = 128
E = 320000

NCORES = 2
NSUB = 16
NW = NCORES * NSUB

NPAD = 10240
ROWS_PER_TILE = NPAD // NSUB
EPW = E // NW
EB = 64
NBC = 16
NCHUNK = 10
NB = NBC * NCHUNK
EPW_PAD = NB * EB
DEGW = 16
BN = 512
GRID = NPAD // BN

_f32 = jnp.float32


def _project_body(h_ref, w_ref, out_ref):
    out_ref[...] = lax.dot_general(
        h_ref[...], w_ref[...], (((0,), (0,)), ((), ())),
        preferred_element_type=_f32)


def _project(h_pad, w):
    return pl.pallas_call(
        _project_body,
        grid=(GRID,),
        in_specs=[
            pl.BlockSpec((D, BN), lambda i: (0, i)),
            pl.BlockSpec((D, D), lambda i: (0, 0)),
        ],
        out_specs=pl.BlockSpec((BN, D), lambda i: (i, 0)),
        out_shape=jax.ShapeDtypeStruct((NPAD, D), _f32),
    )(h_pad, w)


def _deg_inv_block(deg_ref):
    deg = deg_ref[0, :, 0] + deg_ref[1, :, 0]
    return 1.0 / jnp.maximum(deg, 1.0)


def _combine1_body(h_ref, agg_ref, deg_ref, w_self_ref, b_ref, w_next_ref,
                   x1_ref, y2_ref):
    xb = lax.dot_general(h_ref[...], w_self_ref[...], (((0,), (0,)), ((), ())),
                         preferred_element_type=_f32)
    aggb = agg_ref[0] + agg_ref[1]
    dinv = _deg_inv_block(deg_ref)
    x1 = jnp.maximum(xb + aggb * dinv[:, None] + b_ref[0][None, :], 0.0)
    x1_ref[...] = x1
    y2_ref[...] = jnp.dot(x1, w_next_ref[...], preferred_element_type=_f32)


def _combine1(h_pad, agg, deg, w_self, b, w_next):
    return pl.pallas_call(
        _combine1_body,
        grid=(GRID,),
        in_specs=[
            pl.BlockSpec((D, BN), lambda i: (0, i)),
            pl.BlockSpec((NCORES, BN, D), lambda i: (0, i, 0)),
            pl.BlockSpec((NCORES, BN, D), lambda i: (0, i, 0)),
            pl.BlockSpec((D, D), lambda i: (0, 0)),
            pl.BlockSpec((1, D), lambda i: (0, 0)),
            pl.BlockSpec((D, D), lambda i: (0, 0)),
        ],
        out_specs=[
            pl.BlockSpec((BN, D), lambda i: (i, 0)),
            pl.BlockSpec((BN, D), lambda i: (i, 0)),
        ],
        out_shape=[
            jax.ShapeDtypeStruct((NPAD, D), _f32),
            jax.ShapeDtypeStruct((NPAD, D), _f32),
        ],
    )(h_pad, agg, deg, w_self, b, w_next)


def _combine2_body(x1_ref, agg_ref, deg_ref, w_self_ref, b_ref, x2_ref):
    xb = jnp.dot(x1_ref[...], w_self_ref[...], preferred_element_type=_f32)
    aggb = agg_ref[0] + agg_ref[1]
    dinv = _deg_inv_block(deg_ref)
    x2_ref[...] = xb + aggb * dinv[:, None] + b_ref[0][None, :]


def _combine2(x1, agg, deg, w_self, b):
    return pl.pallas_call(
        _combine2_body,
        grid=(GRID,),
        in_specs=[
            pl.BlockSpec((BN, D), lambda i: (i, 0)),
            pl.BlockSpec((NCORES, BN, D), lambda i: (0, i, 0)),
            pl.BlockSpec((NCORES, BN, D), lambda i: (0, i, 0)),
            pl.BlockSpec((D, D), lambda i: (0, 0)),
            pl.BlockSpec((1, D), lambda i: (0, 0)),
        ],
        out_specs=pl.BlockSpec((BN, D), lambda i: (i, 0)),
        out_shape=jax.ShapeDtypeStruct((NPAD, D), _f32),
    )(x1, agg, deg, w_self, b)


_sc_mesh = plsc.VectorSubcoreMesh(core_axis_name="c", subcore_axis_name="s",
                                  num_cores=NCORES, num_subcores=NSUB)


def _fill_rows(rows_v, nrows, value16):
    def frow(r, _):
        for j in range(D // 16):
            rows_v[r, pl.ds(j * 16, 16)] = value16
        return 0

    lax.fori_loop(0, nrows, frow, 0)


def _make_sc_scatter():
    scratch = [
        pltpu.VMEM_SHARED((NPAD, D), _f32),
        pltpu.VMEM((NBC, EB), jnp.int32),
        pltpu.VMEM((NBC, EB), jnp.int32),
        pltpu.VMEM((EB, D), _f32),
        pltpu.SemaphoreType.DMA,
    ]

    def body(y_hbm, src_hbm, dst_hbm, agg_hbm, acc_sh, src_v, dst_v, rows_v,
             sem):
        c = lax.axis_index("c")
        s = lax.axis_index("s")
        w = c * NSUB + s
        base = s * ROWS_PER_TILE

        _fill_rows(rows_v, EB, jnp.zeros((16,), _f32))
        for j in range(ROWS_PER_TILE // EB):
            pltpu.sync_copy(rows_v, acc_sh.at[pl.ds(base + j * EB, EB)])

        plsc.subcore_barrier()

        def chunk(k, _):
            pltpu.sync_copy(src_hbm.at[w, pl.ds(k * NBC, NBC)], src_v)
            pltpu.sync_copy(dst_hbm.at[w, pl.ds(k * NBC, NBC)], dst_v)

            def batch(i, _):
                pltpu.async_copy(y_hbm.at[src_v.at[i]], rows_v, sem).wait()
                pltpu.sync_copy(rows_v, acc_sh.at[dst_v.at[i]], add=True)
                return 0

            lax.fori_loop(0, NBC, batch, 0)
            return 0

        lax.fori_loop(0, NCHUNK, chunk, 0)

        plsc.subcore_barrier()

        for j in range(ROWS_PER_TILE // EB):
            o = base + j * EB
            pltpu.sync_copy(acc_sh.at[pl.ds(o, EB)], rows_v)
            pltpu.sync_copy(rows_v, agg_hbm.at[c, pl.ds(o, EB)])

    return pl.kernel(body,
                     out_type=jax.ShapeDtypeStruct((NCORES, NPAD, D), _f32),
                     mesh=_sc_mesh, scratch_types=scratch)


def _make_sc_deg():
    scratch = [
        pltpu.VMEM_SHARED((NPAD, D), _f32),
        pltpu.VMEM((NBC, EB), jnp.int32),
        pltpu.VMEM((EB, D), _f32),
    ]

    def body(dst_hbm, deg_hbm, acc_sh, dst_v, rows_v):
        c = lax.axis_index("c")
        s = lax.axis_index("s")
        w = c * NSUB + s
        base = s * ROWS_PER_TILE

        _fill_rows(rows_v, EB, jnp.zeros((16,), _f32))
        for j in range(ROWS_PER_TILE // EB):
            pltpu.sync_copy(rows_v, acc_sh.at[pl.ds(base + j * EB, EB)])
        _fill_rows(rows_v, EB, jnp.ones((16,), _f32))

        plsc.subcore_barrier()

        def chunk(k, _):
            pltpu.sync_copy(dst_hbm.at[w, pl.ds(k * NBC, NBC)], dst_v)

            def batch(i, _):
                pltpu.sync_copy(rows_v, acc_sh.at[dst_v.at[i]], add=True)
                return 0

            lax.fori_loop(0, NBC, batch, 0)
            return 0

        lax.fori_loop(0, NCHUNK, chunk, 0)

        plsc.subcore_barrier()

        for j in range(ROWS_PER_TILE // EB):
            o = base + j * EB
            pltpu.sync_copy(acc_sh.at[pl.ds(o, EB)], rows_v)
            pltpu.sync_copy(rows_v, deg_hbm.at[c, pl.ds(o, EB)])

    return pl.kernel(body,
                     out_type=jax.ShapeDtypeStruct((NCORES, NPAD, D), _f32),
                     mesh=_sc_mesh, scratch_types=scratch)


_sc_scatter = _make_sc_scatter()
_sc_deg = _make_sc_deg()


@jax.jit
def kernel(h, edge_index, W1_self, W1_neigh, b1, W2_self, W2_neigh, b2):
    h_pad = jnp.pad(h, ((0, 0), (0, NPAD - N)))

    src = edge_index[0].reshape(NW, EPW)
    dst = edge_index[1].reshape(NW, EPW)
    padlen = EPW_PAD - EPW
    pad_idx = (N + (jnp.arange(padlen, dtype=jnp.int32) % 8))[None, :]
    pad_blk = jnp.broadcast_to(pad_idx, (NW, padlen))
    src_p = jnp.concatenate([src, pad_blk], axis=1).reshape(NW, NB, EB)
    dst_p = jnp.concatenate([dst, pad_blk], axis=1).reshape(NW, NB, EB)

    b1r = b1[None, :]
    b2r = b2[None, :]

    y1 = _project(h_pad, W1_neigh)
    agg1 = _sc_scatter(y1, src_p, dst_p)
    deg = _sc_deg(dst_p)
    x1, y2 = _combine1(h_pad, agg1, deg, W1_self, b1r, W2_neigh)
    agg2 = _sc_scatter(y2, src_p, dst_p)
    x2 = _combine2(x1, agg2, deg, W2_self, b2r)
    return x2[:N].T


# speedup vs baseline: 7.2148x; 1.3361x over previous
import functools

import jax
import jax.numpy as jnp
from jax import lax
from jax.experimental import pallas as pl
from jax.experimental.pallas import tpu as pltpu
from jax.experimental.pallas import tpu_sc as plsc

N = 10000
D = 128
E = 320000

NCORES = 2
NSUB = 16
NW = NCORES * NSUB

NPAD = 10240
ROWS_PER_TILE = NPAD // NSUB
EPW = E // NW
EB = 64
NBC = 16
NCHUNK = 10
NB = NBC * NCHUNK
EPW_PAD = NB * EB
DEGW = 16
BN = 512
GRID = NPAD // BN

_f32 = jnp.float32


def _project_body(h_ref, w_ref, out_ref):
    out_ref[...] = lax.dot_general(
        h_ref[...], w_ref[...], (((0,), (0,)), ((), ())),
        preferred_element_type=_f32)


def _project(h_pad, w):
    return pl.pallas_call(
        _project_body,
        grid=(GRID,),
        in_specs=[
            pl.BlockSpec((D, BN), lambda i: (0, i)),
            pl.BlockSpec((D, D), lambda i: (0, 0)),
        ],
        out_specs=pl.BlockSpec((BN, D), lambda i: (i, 0)),
        out_shape=jax.ShapeDtypeStruct((NPAD, D), _f32),
    )(h_pad, w)


def _deg_inv_block(deg_ref):
    deg = deg_ref[0, :, 0] + deg_ref[1, :, 0]
    return 1.0 / jnp.maximum(deg, 1.0)


def _combine1_body(h_ref, agg_ref, deg_ref, w_self_ref, b_ref, w_next_ref,
                   x1_ref, y2_ref):
    xb = lax.dot_general(h_ref[...], w_self_ref[...], (((0,), (0,)), ((), ())),
                         preferred_element_type=_f32)
    aggb = agg_ref[0] + agg_ref[1]
    dinv = _deg_inv_block(deg_ref)
    x1 = jnp.maximum(xb + aggb * dinv[:, None] + b_ref[0][None, :], 0.0)
    x1_ref[...] = x1
    y2_ref[...] = jnp.dot(x1, w_next_ref[...], preferred_element_type=_f32)


def _combine1(h_pad, agg, deg, w_self, b, w_next):
    return pl.pallas_call(
        _combine1_body,
        grid=(GRID,),
        in_specs=[
            pl.BlockSpec((D, BN), lambda i: (0, i)),
            pl.BlockSpec((NCORES, BN, D), lambda i: (0, i, 0)),
            pl.BlockSpec((NCORES, BN, D), lambda i: (0, i, 0)),
            pl.BlockSpec((D, D), lambda i: (0, 0)),
            pl.BlockSpec((1, D), lambda i: (0, 0)),
            pl.BlockSpec((D, D), lambda i: (0, 0)),
        ],
        out_specs=[
            pl.BlockSpec((BN, D), lambda i: (i, 0)),
            pl.BlockSpec((BN, D), lambda i: (i, 0)),
        ],
        out_shape=[
            jax.ShapeDtypeStruct((NPAD, D), _f32),
            jax.ShapeDtypeStruct((NPAD, D), _f32),
        ],
    )(h_pad, agg, deg, w_self, b, w_next)


def _combine2_body(x1_ref, agg_ref, deg_ref, w_self_ref, b_ref, x2_ref):
    xb = jnp.dot(x1_ref[...], w_self_ref[...], preferred_element_type=_f32)
    aggb = agg_ref[0] + agg_ref[1]
    dinv = _deg_inv_block(deg_ref)
    x2_ref[...] = xb + aggb * dinv[:, None] + b_ref[0][None, :]


def _combine2(x1, agg, deg, w_self, b):
    return pl.pallas_call(
        _combine2_body,
        grid=(GRID,),
        in_specs=[
            pl.BlockSpec((BN, D), lambda i: (i, 0)),
            pl.BlockSpec((NCORES, BN, D), lambda i: (0, i, 0)),
            pl.BlockSpec((NCORES, BN, D), lambda i: (0, i, 0)),
            pl.BlockSpec((D, D), lambda i: (0, 0)),
            pl.BlockSpec((1, D), lambda i: (0, 0)),
        ],
        out_specs=pl.BlockSpec((BN, D), lambda i: (i, 0)),
        out_shape=jax.ShapeDtypeStruct((NPAD, D), _f32),
    )(x1, agg, deg, w_self, b)


_sc_mesh = plsc.VectorSubcoreMesh(core_axis_name="c", subcore_axis_name="s",
                                  num_cores=NCORES, num_subcores=NSUB)


def _fill_rows(rows_v, nrows, value16):
    def frow(r, _):
        for j in range(D // 16):
            rows_v[r, pl.ds(j * 16, 16)] = value16
        return 0

    lax.fori_loop(0, nrows, frow, 0)


def _make_sc_scatter():
    scratch = [
        pltpu.VMEM_SHARED((NPAD, D), _f32),
        pltpu.VMEM((NBC, EB), jnp.int32),
        pltpu.VMEM((NBC, EB), jnp.int32),
        pltpu.VMEM((EB, D), _f32),
        pltpu.VMEM((EB, D), _f32),
        pltpu.SemaphoreType.DMA,
        pltpu.SemaphoreType.DMA,
    ]

    def body(y_hbm, src_hbm, dst_hbm, agg_hbm, acc_sh, src_v, dst_v,
             rows0, rows1, sem0, sem1):
        c = lax.axis_index("c")
        s = lax.axis_index("s")
        w = c * NSUB + s
        base = s * ROWS_PER_TILE

        _fill_rows(rows0, EB, jnp.zeros((16,), _f32))
        for j in range(ROWS_PER_TILE // EB):
            pltpu.sync_copy(rows0, acc_sh.at[pl.ds(base + j * EB, EB)])

        plsc.subcore_barrier()

        def gstart(i, buf, sem):
            pltpu.async_copy(y_hbm.at[src_v.at[i]], buf, sem)

        def gwait(buf, sem):
            pltpu.make_async_copy(y_hbm.at[src_v.at[0]], buf, sem).wait()

        def scat(i, buf):
            pltpu.sync_copy(buf, acc_sh.at[dst_v.at[i]], add=True)

        def chunk(k, _):
            pltpu.sync_copy(src_hbm.at[w, pl.ds(k * NBC, NBC)], src_v)
            pltpu.sync_copy(dst_hbm.at[w, pl.ds(k * NBC, NBC)], dst_v)

            gstart(0, rows0, sem0)

            def pair(p, _):
                i = p * 2
                gstart(i + 1, rows1, sem1)
                gwait(rows0, sem0)
                scat(i, rows0)
                gstart(i + 2, rows0, sem0)
                gwait(rows1, sem1)
                scat(i + 1, rows1)
                return 0

            lax.fori_loop(0, NBC // 2 - 1, pair, 0)

            i = NBC - 2
            gstart(i + 1, rows1, sem1)
            gwait(rows0, sem0)
            scat(i, rows0)
            gwait(rows1, sem1)
            scat(i + 1, rows1)
            return 0

        lax.fori_loop(0, NCHUNK, chunk, 0)

        plsc.subcore_barrier()

        for j in range(ROWS_PER_TILE // EB):
            o = base + j * EB
            pltpu.sync_copy(acc_sh.at[pl.ds(o, EB)], rows0)
            pltpu.sync_copy(rows0, agg_hbm.at[c, pl.ds(o, EB)])

    return pl.kernel(body,
                     out_type=jax.ShapeDtypeStruct((NCORES, NPAD, D), _f32),
                     mesh=_sc_mesh, scratch_types=scratch)


def _make_sc_deg():
    scratch = [
        pltpu.VMEM_SHARED((NPAD, D), _f32),
        pltpu.VMEM((NBC, EB), jnp.int32),
        pltpu.VMEM((EB, D), _f32),
    ]

    def body(dst_hbm, deg_hbm, acc_sh, dst_v, rows_v):
        c = lax.axis_index("c")
        s = lax.axis_index("s")
        w = c * NSUB + s
        base = s * ROWS_PER_TILE

        _fill_rows(rows_v, EB, jnp.zeros((16,), _f32))
        for j in range(ROWS_PER_TILE // EB):
            pltpu.sync_copy(rows_v, acc_sh.at[pl.ds(base + j * EB, EB)])
        _fill_rows(rows_v, EB, jnp.ones((16,), _f32))

        plsc.subcore_barrier()

        def chunk(k, _):
            pltpu.sync_copy(dst_hbm.at[w, pl.ds(k * NBC, NBC)], dst_v)

            def batch(i, _):
                pltpu.sync_copy(rows_v, acc_sh.at[dst_v.at[i]], add=True)
                return 0

            lax.fori_loop(0, NBC, batch, 0)
            return 0

        lax.fori_loop(0, NCHUNK, chunk, 0)

        plsc.subcore_barrier()

        for j in range(ROWS_PER_TILE // EB):
            o = base + j * EB
            pltpu.sync_copy(acc_sh.at[pl.ds(o, EB)], rows_v)
            pltpu.sync_copy(rows_v, deg_hbm.at[c, pl.ds(o, EB)])

    return pl.kernel(body,
                     out_type=jax.ShapeDtypeStruct((NCORES, NPAD, D), _f32),
                     mesh=_sc_mesh, scratch_types=scratch)


_sc_scatter = _make_sc_scatter()
_sc_deg = _make_sc_deg()


@jax.jit
def kernel(h, edge_index, W1_self, W1_neigh, b1, W2_self, W2_neigh, b2):
    h_pad = jnp.pad(h, ((0, 0), (0, NPAD - N)))

    src = edge_index[0].reshape(NW, EPW)
    dst = edge_index[1].reshape(NW, EPW)
    padlen = EPW_PAD - EPW
    pad_idx = (N + (jnp.arange(padlen, dtype=jnp.int32) % 8))[None, :]
    pad_blk = jnp.broadcast_to(pad_idx, (NW, padlen))
    src_p = jnp.concatenate([src, pad_blk], axis=1).reshape(NW, NB, EB)
    dst_p = jnp.concatenate([dst, pad_blk], axis=1).reshape(NW, NB, EB)

    b1r = b1[None, :]
    b2r = b2[None, :]

    y1 = _project(h_pad, W1_neigh)
    agg1 = _sc_scatter(y1, src_p, dst_p)
    deg = _sc_deg(dst_p)
    x1, y2 = _combine1(h_pad, agg1, deg, W1_self, b1r, W2_neigh)
    agg2 = _sc_scatter(y2, src_p, dst_p)
    x2 = _combine2(x1, agg2, deg, W2_self, b2r)
    return x2[:N].T


# speedup vs baseline: 7.8505x; 1.0881x over previous
import functools

import jax
import jax.numpy as jnp
from jax import lax
from jax.experimental import pallas as pl
from jax.experimental.pallas import tpu as pltpu
from jax.experimental.pallas import tpu_sc as plsc

N = 10000
D = 128
E = 320000

NCORES = 2
NSUB = 16
NW = NCORES * NSUB

NPAD = 10240
ROWS_PER_TILE = NPAD // NSUB
EPW = E // NW
EB = 64
NBC = 16
NCHUNK = 10
NB = NBC * NCHUNK
EPW_PAD = NB * EB
DEGW = 16
BN = 512
GRID = NPAD // BN

_f32 = jnp.float32


def _project_body(h_ref, w_ref, out_ref):
    out_ref[...] = lax.dot_general(
        h_ref[...], w_ref[...], (((0,), (0,)), ((), ())),
        preferred_element_type=_f32)


def _project(h_pad, w):
    return pl.pallas_call(
        _project_body,
        grid=(GRID,),
        in_specs=[
            pl.BlockSpec((D, BN), lambda i: (0, i)),
            pl.BlockSpec((D, D), lambda i: (0, 0)),
        ],
        out_specs=pl.BlockSpec((BN, D), lambda i: (i, 0)),
        out_shape=jax.ShapeDtypeStruct((NPAD, D), _f32),
    )(h_pad, w)


def _deg_inv_block(deg_ref):
    deg = deg_ref[0, :, 0] + deg_ref[1, :, 0]
    return 1.0 / jnp.maximum(deg, 1.0)


def _combine1_body(h_ref, agg_ref, deg_ref, w_self_ref, b_ref, w_next_ref,
                   x1_ref, y2_ref):
    xb = lax.dot_general(h_ref[...], w_self_ref[...], (((0,), (0,)), ((), ())),
                         preferred_element_type=_f32)
    aggb = agg_ref[0] + agg_ref[1]
    dinv = _deg_inv_block(deg_ref)
    x1 = jnp.maximum(xb + aggb * dinv[:, None] + b_ref[0][None, :], 0.0)
    x1_ref[...] = x1
    y2_ref[...] = jnp.dot(x1, w_next_ref[...], preferred_element_type=_f32)


def _combine1(h_pad, agg, deg, w_self, b, w_next):
    return pl.pallas_call(
        _combine1_body,
        grid=(GRID,),
        in_specs=[
            pl.BlockSpec((D, BN), lambda i: (0, i)),
            pl.BlockSpec((NCORES, BN, D), lambda i: (0, i, 0)),
            pl.BlockSpec((NCORES, BN, D), lambda i: (0, i, 0)),
            pl.BlockSpec((D, D), lambda i: (0, 0)),
            pl.BlockSpec((1, D), lambda i: (0, 0)),
            pl.BlockSpec((D, D), lambda i: (0, 0)),
        ],
        out_specs=[
            pl.BlockSpec((BN, D), lambda i: (i, 0)),
            pl.BlockSpec((BN, D), lambda i: (i, 0)),
        ],
        out_shape=[
            jax.ShapeDtypeStruct((NPAD, D), _f32),
            jax.ShapeDtypeStruct((NPAD, D), _f32),
        ],
    )(h_pad, agg, deg, w_self, b, w_next)


def _combine2_body(x1_ref, agg_ref, deg_ref, w_self_ref, b_ref, x2_ref):
    xb = jnp.dot(x1_ref[...], w_self_ref[...], preferred_element_type=_f32)
    aggb = agg_ref[0] + agg_ref[1]
    dinv = _deg_inv_block(deg_ref)
    x2_ref[...] = xb + aggb * dinv[:, None] + b_ref[0][None, :]


def _combine2(x1, agg, deg, w_self, b):
    return pl.pallas_call(
        _combine2_body,
        grid=(GRID,),
        in_specs=[
            pl.BlockSpec((BN, D), lambda i: (i, 0)),
            pl.BlockSpec((NCORES, BN, D), lambda i: (0, i, 0)),
            pl.BlockSpec((NCORES, BN, D), lambda i: (0, i, 0)),
            pl.BlockSpec((D, D), lambda i: (0, 0)),
            pl.BlockSpec((1, D), lambda i: (0, 0)),
        ],
        out_specs=pl.BlockSpec((BN, D), lambda i: (i, 0)),
        out_shape=jax.ShapeDtypeStruct((NPAD, D), _f32),
    )(x1, agg, deg, w_self, b)


_sc_mesh = plsc.VectorSubcoreMesh(core_axis_name="c", subcore_axis_name="s",
                                  num_cores=NCORES, num_subcores=NSUB)


def _fill_rows(rows_v, nrows, value16):
    def frow(r, _):
        for j in range(D // 16):
            rows_v[r, pl.ds(j * 16, 16)] = value16
        return 0

    lax.fori_loop(0, nrows, frow, 0)


def _make_sc_scatter():
    scratch = [
        pltpu.VMEM_SHARED((NPAD, D), _f32),
        pltpu.VMEM((NBC, EB), jnp.int32),
        pltpu.VMEM((NBC, EB), jnp.int32),
        pltpu.VMEM((EB, D), _f32),
        pltpu.VMEM((EB, D), _f32),
        pltpu.VMEM((EB, D), _f32),
        pltpu.SemaphoreType.DMA,
        pltpu.SemaphoreType.DMA,
        pltpu.SemaphoreType.DMA,
    ]

    def body(y_hbm, src_hbm, dst_hbm, agg_hbm, acc_sh, src_v, dst_v,
             rows0, rows1, rows2, sem0, sem1, sem2):
        c = lax.axis_index("c")
        s = lax.axis_index("s")
        w = c * NSUB + s
        base = s * ROWS_PER_TILE

        _fill_rows(rows0, EB, jnp.zeros((16,), _f32))
        for j in range(ROWS_PER_TILE // EB):
            pltpu.sync_copy(rows0, acc_sh.at[pl.ds(base + j * EB, EB)])

        plsc.subcore_barrier()

        def gstart(i, buf, sem):
            pltpu.async_copy(y_hbm.at[src_v.at[i]], buf, sem)

        def gwait(buf, sem):
            pltpu.make_async_copy(y_hbm.at[src_v.at[0]], buf, sem).wait()

        def scat(i, buf):
            pltpu.sync_copy(buf, acc_sh.at[dst_v.at[i]], add=True)

        def chunk(k, _):
            pltpu.sync_copy(src_hbm.at[w, pl.ds(k * NBC, NBC)], src_v)
            pltpu.sync_copy(dst_hbm.at[w, pl.ds(k * NBC, NBC)], dst_v)

            gstart(0, rows0, sem0)
            gstart(1, rows1, sem1)

            def tri(t, _):
                i = t * 3
                gstart(i + 2, rows2, sem2)
                gwait(rows0, sem0)
                scat(i, rows0)
                gstart(i + 3, rows0, sem0)
                gwait(rows1, sem1)
                scat(i + 1, rows1)
                gstart(i + 4, rows1, sem1)
                gwait(rows2, sem2)
                scat(i + 2, rows2)
                return 0

            lax.fori_loop(0, (NBC - 4) // 3, tri, 0)

            i = NBC - 4
            gstart(i + 2, rows2, sem2)
            gwait(rows0, sem0)
            scat(i, rows0)
            gstart(i + 3, rows0, sem0)
            gwait(rows1, sem1)
            scat(i + 1, rows1)
            gwait(rows2, sem2)
            scat(i + 2, rows2)
            gwait(rows0, sem0)
            scat(i + 3, rows0)
            return 0

        lax.fori_loop(0, NCHUNK, chunk, 0)

        plsc.subcore_barrier()

        for j in range(ROWS_PER_TILE // EB):
            o = base + j * EB
            pltpu.sync_copy(acc_sh.at[pl.ds(o, EB)], rows0)
            pltpu.sync_copy(rows0, agg_hbm.at[c, pl.ds(o, EB)])

    return pl.kernel(body,
                     out_type=jax.ShapeDtypeStruct((NCORES, NPAD, D), _f32),
                     mesh=_sc_mesh, scratch_types=scratch)


def _make_sc_deg():
    scratch = [
        pltpu.VMEM_SHARED((NPAD, D), _f32),
        pltpu.VMEM((NBC, EB), jnp.int32),
        pltpu.VMEM((EB, D), _f32),
    ]

    def body(dst_hbm, deg_hbm, acc_sh, dst_v, rows_v):
        c = lax.axis_index("c")
        s = lax.axis_index("s")
        w = c * NSUB + s
        base = s * ROWS_PER_TILE

        _fill_rows(rows_v, EB, jnp.zeros((16,), _f32))
        for j in range(ROWS_PER_TILE // EB):
            pltpu.sync_copy(rows_v, acc_sh.at[pl.ds(base + j * EB, EB)])
        _fill_rows(rows_v, EB, jnp.ones((16,), _f32))

        plsc.subcore_barrier()

        def chunk(k, _):
            pltpu.sync_copy(dst_hbm.at[w, pl.ds(k * NBC, NBC)], dst_v)

            def batch(i, _):
                pltpu.sync_copy(rows_v, acc_sh.at[dst_v.at[i]], add=True)
                return 0

            lax.fori_loop(0, NBC, batch, 0)
            return 0

        lax.fori_loop(0, NCHUNK, chunk, 0)

        plsc.subcore_barrier()

        for j in range(ROWS_PER_TILE // EB):
            o = base + j * EB
            pltpu.sync_copy(acc_sh.at[pl.ds(o, EB)], rows_v)
            pltpu.sync_copy(rows_v, deg_hbm.at[c, pl.ds(o, EB)])

    return pl.kernel(body,
                     out_type=jax.ShapeDtypeStruct((NCORES, NPAD, D), _f32),
                     mesh=_sc_mesh, scratch_types=scratch)


_sc_scatter = _make_sc_scatter()
_sc_deg = _make_sc_deg()


@jax.jit
def kernel(h, edge_index, W1_self, W1_neigh, b1, W2_self, W2_neigh, b2):
    h_pad = jnp.pad(h, ((0, 0), (0, NPAD - N)))

    src = edge_index[0].reshape(NW, EPW)
    dst = edge_index[1].reshape(NW, EPW)
    padlen = EPW_PAD - EPW
    pad_idx = (N + (jnp.arange(padlen, dtype=jnp.int32) % 8))[None, :]
    pad_blk = jnp.broadcast_to(pad_idx, (NW, padlen))
    src_p = jnp.concatenate([src, pad_blk], axis=1).reshape(NW, NB, EB)
    dst_p = jnp.concatenate([dst, pad_blk], axis=1).reshape(NW, NB, EB)

    b1r = b1[None, :]
    b2r = b2[None, :]

    y1 = _project(h_pad, W1_neigh)
    agg1 = _sc_scatter(y1, src_p, dst_p)
    deg = _sc_deg(dst_p)
    x1, y2 = _combine1(h_pad, agg1, deg, W1_self, b1r, W2_neigh)
    agg2 = _sc_scatter(y2, src_p, dst_p)
    x2 = _combine2(x1, agg2, deg, W2_self, b2r)
    return x2[:N].T


# speedup vs baseline: 8.3993x; 1.0699x over previous
import functools

import jax
import jax.numpy as jnp
from jax import lax
from jax.experimental import pallas as pl
from jax.experimental.pallas import tpu as pltpu
from jax.experimental.pallas import tpu_sc as plsc

N = 10000
D = 128
E = 320000

NCORES = 2
NSUB = 16
NW = NCORES * NSUB

NPAD = 10240
ROWS_PER_TILE = NPAD // NSUB
EPW = E // NW
EB = 64
NBC = 40
NCHUNK = 4
NB = NBC * NCHUNK
EPW_PAD = NB * EB
BN = 512
GRID = NPAD // BN

_f32 = jnp.float32


def _project_body(h_ref, w_ref, out_ref):
    out_ref[...] = lax.dot_general(
        h_ref[...], w_ref[...], (((0,), (0,)), ((), ())),
        preferred_element_type=_f32)


def _project(h, w):
    return pl.pallas_call(
        _project_body,
        grid=(GRID,),
        in_specs=[
            pl.BlockSpec((D, BN), lambda i: (0, i)),
            pl.BlockSpec((D, D), lambda i: (0, 0)),
        ],
        out_specs=pl.BlockSpec((BN, D), lambda i: (i, 0)),
        out_shape=jax.ShapeDtypeStruct((NPAD, D), _f32),
    )(h, w)


def _combine1_body(h_ref, agg_ref, deg_ref, w_self_ref, b_ref, w_next_ref,
                   x1_ref, y2_ref, dinv_ref):
    xb = lax.dot_general(h_ref[...], w_self_ref[...], (((0,), (0,)), ((), ())),
                         preferred_element_type=_f32)
    aggb = agg_ref[0] + agg_ref[1]
    deg = deg_ref[0, :, 0] + deg_ref[1, :, 0]
    dinv = 1.0 / jnp.maximum(deg, 1.0)
    x1 = jnp.maximum(xb + aggb * dinv[:, None] + b_ref[0][None, :], 0.0)
    x1_ref[...] = x1
    y2_ref[...] = jnp.dot(x1, w_next_ref[...], preferred_element_type=_f32)
    dinv_ref[...] = dinv[None, :]


def _combine1(h, agg, deg, w_self, b, w_next):
    return pl.pallas_call(
        _combine1_body,
        grid=(GRID,),
        in_specs=[
            pl.BlockSpec((D, BN), lambda i: (0, i)),
            pl.BlockSpec((NCORES, BN, D), lambda i: (0, i, 0)),
            pl.BlockSpec((NCORES, BN, D), lambda i: (0, i, 0)),
            pl.BlockSpec((D, D), lambda i: (0, 0)),
            pl.BlockSpec((1, D), lambda i: (0, 0)),
            pl.BlockSpec((D, D), lambda i: (0, 0)),
        ],
        out_specs=[
            pl.BlockSpec((BN, D), lambda i: (i, 0)),
            pl.BlockSpec((BN, D), lambda i: (i, 0)),
            pl.BlockSpec((1, BN), lambda i: (0, i)),
        ],
        out_shape=[
            jax.ShapeDtypeStruct((NPAD, D), _f32),
            jax.ShapeDtypeStruct((NPAD, D), _f32),
            jax.ShapeDtypeStruct((1, NPAD), _f32),
        ],
    )(h, agg, deg, w_self, b, w_next)


def _combine2_body(x1_ref, agg_ref, dinv_ref, w_self_ref, b_ref, x2t_ref):
    xb = jnp.dot(x1_ref[...], w_self_ref[...], preferred_element_type=_f32)
    aggb = agg_ref[0] + agg_ref[1]
    dinv = dinv_ref[0]
    x2 = xb + aggb * dinv[:, None] + b_ref[0][None, :]
    x2t_ref[...] = x2.T


def _combine2(x1, agg, dinv, w_self, b):
    return pl.pallas_call(
        _combine2_body,
        grid=(GRID,),
        in_specs=[
            pl.BlockSpec((BN, D), lambda i: (i, 0)),
            pl.BlockSpec((NCORES, BN, D), lambda i: (0, i, 0)),
            pl.BlockSpec((1, BN), lambda i: (0, i)),
            pl.BlockSpec((D, D), lambda i: (0, 0)),
            pl.BlockSpec((1, D), lambda i: (0, 0)),
        ],
        out_specs=pl.BlockSpec((D, BN), lambda i: (0, i)),
        out_shape=jax.ShapeDtypeStruct((D, NPAD), _f32),
    )(x1, agg, dinv, w_self, b)


_sc_mesh = plsc.VectorSubcoreMesh(core_axis_name="c", subcore_axis_name="s",
                                  num_cores=NCORES, num_subcores=NSUB)


def _fill_rows(rows_v, nrows, value16):
    def frow(r, _):
        for j in range(D // 16):
            rows_v[r, pl.ds(j * 16, 16)] = value16
        return 0

    lax.fori_loop(0, nrows, frow, 0)


def _make_sc_scatter(with_deg):
    scratch = [
        pltpu.VMEM_SHARED((NPAD, D), _f32),
        pltpu.VMEM((NBC, EB), jnp.int32),
        pltpu.VMEM((NBC, EB), jnp.int32),
        pltpu.VMEM((EB, D), _f32),
        pltpu.VMEM((EB, D), _f32),
        pltpu.VMEM((EB, D), _f32),
        pltpu.SemaphoreType.DMA,
        pltpu.SemaphoreType.DMA,
        pltpu.SemaphoreType.DMA,
    ]

    def body(*refs):
        if with_deg:
            (y_hbm, src_hbm, dst_hbm, agg_hbm, deg_hbm, acc_sh, src_v, dst_v,
             rows0, rows1, rows2, sem0, sem1, sem2) = refs
        else:
            deg_hbm = None
            (y_hbm, src_hbm, dst_hbm, agg_hbm, acc_sh, src_v, dst_v,
             rows0, rows1, rows2, sem0, sem1, sem2) = refs
        c = lax.axis_index("c")
        s = lax.axis_index("s")
        w = c * NSUB + s
        base = s * ROWS_PER_TILE

        _fill_rows(rows0, EB, jnp.zeros((16,), _f32))
        for j in range(ROWS_PER_TILE // EB):
            pltpu.sync_copy(rows0, acc_sh.at[pl.ds(base + j * EB, EB)])

        plsc.subcore_barrier()

        def gstart(i, buf, sem):
            pltpu.async_copy(y_hbm.at[src_v.at[i]], buf, sem)

        def gwait(buf, sem):
            pltpu.make_async_copy(y_hbm.at[src_v.at[0]], buf, sem).wait()

        def scat(i, buf):
            pltpu.sync_copy(buf, acc_sh.at[dst_v.at[i]], add=True)

        def chunk(k, _):
            pltpu.sync_copy(src_hbm.at[w, pl.ds(k * NBC, NBC)], src_v)
            pltpu.sync_copy(dst_hbm.at[w, pl.ds(k * NBC, NBC)], dst_v)

            gstart(0, rows0, sem0)
            gstart(1, rows1, sem1)

            def tri(t, _):
                i = t * 3
                gstart(i + 2, rows2, sem2)
                gwait(rows0, sem0)
                scat(i, rows0)
                gstart(i + 3, rows0, sem0)
                gwait(rows1, sem1)
                scat(i + 1, rows1)
                gstart(i + 4, rows1, sem1)
                gwait(rows2, sem2)
                scat(i + 2, rows2)
                return 0

            lax.fori_loop(0, (NBC - 4) // 3, tri, 0)

            i = NBC - 4
            gstart(i + 2, rows2, sem2)
            gwait(rows0, sem0)
            scat(i, rows0)
            gstart(i + 3, rows0, sem0)
            gwait(rows1, sem1)
            scat(i + 1, rows1)
            gwait(rows2, sem2)
            scat(i + 2, rows2)
            gwait(rows0, sem0)
            scat(i + 3, rows0)
            return 0

        lax.fori_loop(0, NCHUNK, chunk, 0)

        plsc.subcore_barrier()

        for j in range(ROWS_PER_TILE // EB):
            o = base + j * EB
            pltpu.sync_copy(acc_sh.at[pl.ds(o, EB)], rows0)
            pltpu.sync_copy(rows0, agg_hbm.at[c, pl.ds(o, EB)])

        if with_deg:
            _fill_rows(rows0, EB, jnp.zeros((16,), _f32))
            plsc.subcore_barrier()
            for j in range(ROWS_PER_TILE // EB):
                pltpu.sync_copy(rows0, acc_sh.at[pl.ds(base + j * EB, EB)])
            _fill_rows(rows1, EB, jnp.ones((16,), _f32))

            plsc.subcore_barrier()

            def dchunk(k, _):
                pltpu.sync_copy(dst_hbm.at[w, pl.ds(k * NBC, NBC)], dst_v)

                def batch(i, _):
                    pltpu.sync_copy(rows1, acc_sh.at[dst_v.at[i]], add=True)
                    return 0

                lax.fori_loop(0, NBC, batch, 0)
                return 0

            lax.fori_loop(0, NCHUNK, dchunk, 0)

            plsc.subcore_barrier()

            for j in range(ROWS_PER_TILE // EB):
                o = base + j * EB
                pltpu.sync_copy(acc_sh.at[pl.ds(o, EB)], rows0)
                pltpu.sync_copy(rows0, deg_hbm.at[c, pl.ds(o, EB)])

    out_type = [jax.ShapeDtypeStruct((NCORES, NPAD, D), _f32)]
    if with_deg:
        out_type.append(jax.ShapeDtypeStruct((NCORES, NPAD, D), _f32))
    return pl.kernel(body, out_type=out_type, mesh=_sc_mesh,
                     scratch_types=scratch)


_sc_scatter_deg = _make_sc_scatter(with_deg=True)
_sc_scatter = _make_sc_scatter(with_deg=False)


@jax.jit
def kernel(h, edge_index, W1_self, W1_neigh, b1, W2_self, W2_neigh, b2):
    src = edge_index[0].reshape(NW, EPW)
    dst = edge_index[1].reshape(NW, EPW)
    padlen = EPW_PAD - EPW
    pad_idx = (N + (jnp.arange(padlen, dtype=jnp.int32) % 8))[None, :]
    pad_blk = jnp.broadcast_to(pad_idx, (NW, padlen))
    src_p = jnp.concatenate([src, pad_blk], axis=1).reshape(NW, NB, EB)
    dst_p = jnp.concatenate([dst, pad_blk], axis=1).reshape(NW, NB, EB)

    b1r = b1[None, :]
    b2r = b2[None, :]

    y1 = _project(h, W1_neigh)
    agg1, deg = _sc_scatter_deg(y1, src_p, dst_p)
    x1, y2, dinv = _combine1(h, agg1, deg, W1_self, b1r, W2_neigh)
    (agg2,) = _sc_scatter(y2, src_p, dst_p)
    x2t = _combine2(x1, agg2, dinv, W2_self, b2r)
    return x2t[:, :N]


# speedup vs baseline: 8.5346x; 1.0161x over previous
import functools

import jax
import jax.numpy as jnp
from jax import lax
from jax.experimental import pallas as pl
from jax.experimental.pallas import tpu as pltpu
from jax.experimental.pallas import tpu_sc as plsc

N = 10000
D = 128
E = 320000

NCORES = 2
NSUB = 16
NW = NCORES * NSUB

NPAD = 10240
ROWS_PER_TILE = NPAD // NSUB
EPW = E // NW
EB = 128
NBC = 40
NCHUNK = 2
NB = NBC * NCHUNK
EPW_PAD = NB * EB
BN = 512
GRID = NPAD // BN

_f32 = jnp.float32


def _project_body(h_ref, w_ref, out_ref):
    out_ref[...] = lax.dot_general(
        h_ref[...], w_ref[...], (((0,), (0,)), ((), ())),
        preferred_element_type=_f32)


def _project(h, w):
    return pl.pallas_call(
        _project_body,
        grid=(GRID,),
        in_specs=[
            pl.BlockSpec((D, BN), lambda i: (0, i)),
            pl.BlockSpec((D, D), lambda i: (0, 0)),
        ],
        out_specs=pl.BlockSpec((BN, D), lambda i: (i, 0)),
        out_shape=jax.ShapeDtypeStruct((NPAD, D), _f32),
    )(h, w)


def _combine1_body(h_ref, agg_ref, deg_ref, w_self_ref, b_ref, w_next_ref,
                   x1_ref, y2_ref, dinv_ref):
    xb = lax.dot_general(h_ref[...], w_self_ref[...], (((0,), (0,)), ((), ())),
                         preferred_element_type=_f32)
    aggb = agg_ref[0] + agg_ref[1]
    deg = deg_ref[0, :, 0] + deg_ref[1, :, 0]
    dinv = 1.0 / jnp.maximum(deg, 1.0)
    x1 = jnp.maximum(xb + aggb * dinv[:, None] + b_ref[0][None, :], 0.0)
    x1_ref[...] = x1
    y2_ref[...] = jnp.dot(x1, w_next_ref[...], preferred_element_type=_f32)
    dinv_ref[...] = dinv[None, :]


def _combine1(h, agg, deg, w_self, b, w_next):
    return pl.pallas_call(
        _combine1_body,
        grid=(GRID,),
        in_specs=[
            pl.BlockSpec((D, BN), lambda i: (0, i)),
            pl.BlockSpec((NCORES, BN, D), lambda i: (0, i, 0)),
            pl.BlockSpec((NCORES, BN, D), lambda i: (0, i, 0)),
            pl.BlockSpec((D, D), lambda i: (0, 0)),
            pl.BlockSpec((1, D), lambda i: (0, 0)),
            pl.BlockSpec((D, D), lambda i: (0, 0)),
        ],
        out_specs=[
            pl.BlockSpec((BN, D), lambda i: (i, 0)),
            pl.BlockSpec((BN, D), lambda i: (i, 0)),
            pl.BlockSpec((1, BN), lambda i: (0, i)),
        ],
        out_shape=[
            jax.ShapeDtypeStruct((NPAD, D), _f32),
            jax.ShapeDtypeStruct((NPAD, D), _f32),
            jax.ShapeDtypeStruct((1, NPAD), _f32),
        ],
    )(h, agg, deg, w_self, b, w_next)


def _combine2_body(x1_ref, agg_ref, dinv_ref, w_self_ref, b_ref, x2t_ref):
    xb = jnp.dot(x1_ref[...], w_self_ref[...], preferred_element_type=_f32)
    aggb = agg_ref[0] + agg_ref[1]
    dinv = dinv_ref[0]
    x2 = xb + aggb * dinv[:, None] + b_ref[0][None, :]
    x2t_ref[...] = x2.T


def _combine2(x1, agg, dinv, w_self, b):
    return pl.pallas_call(
        _combine2_body,
        grid=(GRID,),
        in_specs=[
            pl.BlockSpec((BN, D), lambda i: (i, 0)),
            pl.BlockSpec((NCORES, BN, D), lambda i: (0, i, 0)),
            pl.BlockSpec((1, BN), lambda i: (0, i)),
            pl.BlockSpec((D, D), lambda i: (0, 0)),
            pl.BlockSpec((1, D), lambda i: (0, 0)),
        ],
        out_specs=pl.BlockSpec((D, BN), lambda i: (0, i)),
        out_shape=jax.ShapeDtypeStruct((D, NPAD), _f32),
    )(x1, agg, dinv, w_self, b)


_sc_mesh = plsc.VectorSubcoreMesh(core_axis_name="c", subcore_axis_name="s",
                                  num_cores=NCORES, num_subcores=NSUB)


def _fill_rows(rows_v, nrows, value16):
    def frow(r, _):
        for j in range(D // 16):
            rows_v[r, pl.ds(j * 16, 16)] = value16
        return 0

    lax.fori_loop(0, nrows, frow, 0)


def _make_sc_scatter(with_deg):
    scratch = [
        pltpu.VMEM_SHARED((NPAD, D), _f32),
        pltpu.VMEM((NBC, EB), jnp.int32),
        pltpu.VMEM((NBC, EB), jnp.int32),
        pltpu.VMEM((EB, D), _f32),
        pltpu.VMEM((EB, D), _f32),
        pltpu.SemaphoreType.DMA,
        pltpu.SemaphoreType.DMA,
    ]

    def body(*refs):
        if with_deg:
            (y_hbm, src_hbm, dst_hbm, agg_hbm, deg_hbm, acc_sh, src_v, dst_v,
             rows0, rows1, sem0, sem1) = refs
        else:
            deg_hbm = None
            (y_hbm, src_hbm, dst_hbm, agg_hbm, acc_sh, src_v, dst_v,
             rows0, rows1, sem0, sem1) = refs
        c = lax.axis_index("c")
        s = lax.axis_index("s")
        w = c * NSUB + s
        base = s * ROWS_PER_TILE

        _fill_rows(rows0, EB, jnp.zeros((16,), _f32))
        for j in range(ROWS_PER_TILE // EB):
            pltpu.sync_copy(rows0, acc_sh.at[pl.ds(base + j * EB, EB)])

        plsc.subcore_barrier()

        def gstart(i, buf, sem):
            pltpu.async_copy(y_hbm.at[src_v.at[i]], buf, sem)

        def gwait(buf, sem):
            pltpu.make_async_copy(y_hbm.at[src_v.at[0]], buf, sem).wait()

        def scat(i, buf):
            pltpu.sync_copy(buf, acc_sh.at[dst_v.at[i]], add=True)

        def chunk(k, _):
            pltpu.sync_copy(src_hbm.at[w, pl.ds(k * NBC, NBC)], src_v)
            pltpu.sync_copy(dst_hbm.at[w, pl.ds(k * NBC, NBC)], dst_v)

            gstart(0, rows0, sem0)

            def pair(p, _):
                i = p * 2
                gstart(i + 1, rows1, sem1)
                gwait(rows0, sem0)
                scat(i, rows0)
                gstart(i + 2, rows0, sem0)
                gwait(rows1, sem1)
                scat(i + 1, rows1)
                return 0

            lax.fori_loop(0, NBC // 2 - 1, pair, 0)

            i = NBC - 2
            gstart(i + 1, rows1, sem1)
            gwait(rows0, sem0)
            scat(i, rows0)
            gwait(rows1, sem1)
            scat(i + 1, rows1)
            return 0

        lax.fori_loop(0, NCHUNK, chunk, 0)

        plsc.subcore_barrier()

        for j in range(ROWS_PER_TILE // EB):
            o = base + j * EB
            pltpu.sync_copy(acc_sh.at[pl.ds(o, EB)], rows0)
            pltpu.sync_copy(rows0, agg_hbm.at[c, pl.ds(o, EB)])

        if with_deg:
            _fill_rows(rows0, EB, jnp.zeros((16,), _f32))
            plsc.subcore_barrier()
            for j in range(ROWS_PER_TILE // EB):
                pltpu.sync_copy(rows0, acc_sh.at[pl.ds(base + j * EB, EB)])
            _fill_rows(rows1, EB, jnp.ones((16,), _f32))

            plsc.subcore_barrier()

            def dchunk(k, _):
                pltpu.sync_copy(dst_hbm.at[w, pl.ds(k * NBC, NBC)], dst_v)

                def batch(i, _):
                    pltpu.sync_copy(rows1, acc_sh.at[dst_v.at[i]], add=True)
                    return 0

                lax.fori_loop(0, NBC, batch, 0)
                return 0

            lax.fori_loop(0, NCHUNK, dchunk, 0)

            plsc.subcore_barrier()

            for j in range(ROWS_PER_TILE // EB):
                o = base + j * EB
                pltpu.sync_copy(acc_sh.at[pl.ds(o, EB)], rows0)
                pltpu.sync_copy(rows0, deg_hbm.at[c, pl.ds(o, EB)])

    out_type = [jax.ShapeDtypeStruct((NCORES, NPAD, D), _f32)]
    if with_deg:
        out_type.append(jax.ShapeDtypeStruct((NCORES, NPAD, D), _f32))
    return pl.kernel(body, out_type=out_type, mesh=_sc_mesh,
                     scratch_types=scratch)


_sc_scatter_deg = _make_sc_scatter(with_deg=True)
_sc_scatter = _make_sc_scatter(with_deg=False)


@jax.jit
def kernel(h, edge_index, W1_self, W1_neigh, b1, W2_self, W2_neigh, b2):
    src = edge_index[0].reshape(NW, EPW)
    dst = edge_index[1].reshape(NW, EPW)
    padlen = EPW_PAD - EPW
    pad_idx = (N + (jnp.arange(padlen, dtype=jnp.int32) % 8))[None, :]
    pad_blk = jnp.broadcast_to(pad_idx, (NW, padlen))
    src_p = jnp.concatenate([src, pad_blk], axis=1).reshape(NW, NB, EB)
    dst_p = jnp.concatenate([dst, pad_blk], axis=1).reshape(NW, NB, EB)

    b1r = b1[None, :]
    b2r = b2[None, :]

    y1 = _project(h, W1_neigh)
    agg1, deg = _sc_scatter_deg(y1, src_p, dst_p)
    x1, y2, dinv = _combine1(h, agg1, deg, W1_self, b1r, W2_neigh)
    (agg2,) = _sc_scatter(y2, src_p, dst_p)
    x2t = _combine2(x1, agg2, dinv, W2_self, b2r)
    return x2t[:, :N]


# speedup vs baseline: 8.5506x; 1.0019x over previous
import functools

import jax
import jax.numpy as jnp
from jax import lax
from jax.experimental import pallas as pl
from jax.experimental.pallas import tpu as pltpu
from jax.experimental.pallas import tpu_sc as plsc

N = 10000
D = 128
E = 320000

NCORES = 2
NSUB = 16
NW = NCORES * NSUB

NPAD = 10240
ROWS_PER_TILE = NPAD // NSUB
EPW = E // NW
EB = 64
NBC = 40
NCHUNK = 4
NB = NBC * NCHUNK
EPW_PAD = NB * EB
BN = 512
GRID = NPAD // BN

_f32 = jnp.float32


def _project_body(h_ref, w_ref, out_ref):
    out_ref[...] = lax.dot_general(
        h_ref[...], w_ref[...], (((0,), (0,)), ((), ())),
        preferred_element_type=_f32)


def _project(h, w):
    return pl.pallas_call(
        _project_body,
        grid=(GRID,),
        in_specs=[
            pl.BlockSpec((D, BN), lambda i: (0, i)),
            pl.BlockSpec((D, D), lambda i: (0, 0)),
        ],
        out_specs=pl.BlockSpec((BN, D), lambda i: (i, 0)),
        out_shape=jax.ShapeDtypeStruct((NPAD, D), _f32),
    )(h, w)


def _combine1_body(h_ref, agg_ref, deg_ref, w_self_ref, b_ref, w_next_ref,
                   x1_ref, y2_ref, dinv_ref):
    xb = lax.dot_general(h_ref[...], w_self_ref[...], (((0,), (0,)), ((), ())),
                         preferred_element_type=_f32)
    aggb = agg_ref[0] + agg_ref[1]
    deg = deg_ref[0, :, 0] + deg_ref[1, :, 0]
    dinv = 1.0 / jnp.maximum(deg, 1.0)
    x1 = jnp.maximum(xb + aggb * dinv[:, None] + b_ref[0][None, :], 0.0)
    x1_ref[...] = x1
    y2_ref[...] = jnp.dot(x1, w_next_ref[...], preferred_element_type=_f32)
    dinv_ref[...] = dinv[None, :]


def _combine1(h, agg, deg, w_self, b, w_next):
    return pl.pallas_call(
        _combine1_body,
        grid=(GRID,),
        in_specs=[
            pl.BlockSpec((D, BN), lambda i: (0, i)),
            pl.BlockSpec((NCORES, BN, D), lambda i: (0, i, 0)),
            pl.BlockSpec((NCORES, BN, D), lambda i: (0, i, 0)),
            pl.BlockSpec((D, D), lambda i: (0, 0)),
            pl.BlockSpec((1, D), lambda i: (0, 0)),
            pl.BlockSpec((D, D), lambda i: (0, 0)),
        ],
        out_specs=[
            pl.BlockSpec((BN, D), lambda i: (i, 0)),
            pl.BlockSpec((BN, D), lambda i: (i, 0)),
            pl.BlockSpec((1, BN), lambda i: (0, i)),
        ],
        out_shape=[
            jax.ShapeDtypeStruct((NPAD, D), _f32),
            jax.ShapeDtypeStruct((NPAD, D), _f32),
            jax.ShapeDtypeStruct((1, NPAD), _f32),
        ],
    )(h, agg, deg, w_self, b, w_next)


def _combine2_body(x1_ref, agg_ref, dinv_ref, w_self_ref, b_ref, x2t_ref):
    xb = jnp.dot(x1_ref[...], w_self_ref[...], preferred_element_type=_f32)
    aggb = agg_ref[0] + agg_ref[1]
    dinv = dinv_ref[0]
    x2 = xb + aggb * dinv[:, None] + b_ref[0][None, :]
    x2t_ref[...] = x2.T


def _combine2(x1, agg, dinv, w_self, b):
    return pl.pallas_call(
        _combine2_body,
        grid=(GRID,),
        in_specs=[
            pl.BlockSpec((BN, D), lambda i: (i, 0)),
            pl.BlockSpec((NCORES, BN, D), lambda i: (0, i, 0)),
            pl.BlockSpec((1, BN), lambda i: (0, i)),
            pl.BlockSpec((D, D), lambda i: (0, 0)),
            pl.BlockSpec((1, D), lambda i: (0, 0)),
        ],
        out_specs=pl.BlockSpec((D, BN), lambda i: (0, i)),
        out_shape=jax.ShapeDtypeStruct((D, NPAD), _f32),
    )(x1, agg, dinv, w_self, b)


_sc_mesh = plsc.VectorSubcoreMesh(core_axis_name="c", subcore_axis_name="s",
                                  num_cores=NCORES, num_subcores=NSUB)


def _fill_rows(rows_v, nrows, value16):
    def frow(r, _):
        for j in range(D // 16):
            rows_v[r, pl.ds(j * 16, 16)] = value16
        return 0

    lax.fori_loop(0, nrows, frow, 0)


def _make_sc_scatter(with_deg):
    scratch = [
        pltpu.VMEM_SHARED((NPAD, D), _f32),
        pltpu.VMEM((NBC, EB), jnp.int32),
        pltpu.VMEM((NBC, EB), jnp.int32),
        pltpu.VMEM((EB, D), _f32),
        pltpu.VMEM((EB, D), _f32),
        pltpu.VMEM((EB, D), _f32),
        pltpu.VMEM((EB, D), _f32),
        pltpu.SemaphoreType.DMA,
        pltpu.SemaphoreType.DMA,
        pltpu.SemaphoreType.DMA,
        pltpu.SemaphoreType.DMA,
    ]

    def body(*refs):
        if with_deg:
            (y_hbm, src_hbm, dst_hbm, agg_hbm, deg_hbm, acc_sh, src_v, dst_v,
             rows0, rows1, rows2, rows3, sem0, sem1, sem2, sem3) = refs
        else:
            deg_hbm = None
            (y_hbm, src_hbm, dst_hbm, agg_hbm, acc_sh, src_v, dst_v,
             rows0, rows1, rows2, rows3, sem0, sem1, sem2, sem3) = refs
        c = lax.axis_index("c")
        s = lax.axis_index("s")
        w = c * NSUB + s
        base = s * ROWS_PER_TILE

        _fill_rows(rows0, EB, jnp.zeros((16,), _f32))
        for j in range(ROWS_PER_TILE // EB):
            pltpu.sync_copy(rows0, acc_sh.at[pl.ds(base + j * EB, EB)])

        plsc.subcore_barrier()

        def gstart(i, buf, sem):
            pltpu.async_copy(y_hbm.at[src_v.at[i]], buf, sem)

        def gwait(buf, sem):
            pltpu.make_async_copy(y_hbm.at[src_v.at[0]], buf, sem).wait()

        def scat(i, buf):
            pltpu.sync_copy(buf, acc_sh.at[dst_v.at[i]], add=True)

        def chunk(k, _):
            pltpu.sync_copy(src_hbm.at[w, pl.ds(k * NBC, NBC)], src_v)
            pltpu.sync_copy(dst_hbm.at[w, pl.ds(k * NBC, NBC)], dst_v)

            gstart(0, rows0, sem0)
            gstart(1, rows1, sem1)
            gstart(2, rows2, sem2)

            def quad(t, _):
                i = t * 4
                gstart(i + 3, rows3, sem3)
                gwait(rows0, sem0)
                scat(i, rows0)
                gstart(i + 4, rows0, sem0)
                gwait(rows1, sem1)
                scat(i + 1, rows1)
                gstart(i + 5, rows1, sem1)
                gwait(rows2, sem2)
                scat(i + 2, rows2)
                gstart(i + 6, rows2, sem2)
                gwait(rows3, sem3)
                scat(i + 3, rows3)
                return 0

            lax.fori_loop(0, (NBC - 8) // 4, quad, 0)

            i = NBC - 8
            gstart(i + 3, rows3, sem3)
            gwait(rows0, sem0)
            scat(i, rows0)
            gstart(i + 4, rows0, sem0)
            gwait(rows1, sem1)
            scat(i + 1, rows1)
            gstart(i + 5, rows1, sem1)
            gwait(rows2, sem2)
            scat(i + 2, rows2)
            gstart(i + 6, rows2, sem2)
            gwait(rows3, sem3)
            scat(i + 3, rows3)
            gstart(i + 7, rows3, sem3)
            gwait(rows0, sem0)
            scat(i + 4, rows0)
            gwait(rows1, sem1)
            scat(i + 5, rows1)
            gwait(rows2, sem2)
            scat(i + 6, rows2)
            gwait(rows3, sem3)
            scat(i + 7, rows3)
            return 0

        lax.fori_loop(0, NCHUNK, chunk, 0)

        plsc.subcore_barrier()

        for j in range(ROWS_PER_TILE // EB):
            o = base + j * EB
            pltpu.sync_copy(acc_sh.at[pl.ds(o, EB)], rows0)
            pltpu.sync_copy(rows0, agg_hbm.at[c, pl.ds(o, EB)])

        if with_deg:
            _fill_rows(rows0, EB, jnp.zeros((16,), _f32))
            plsc.subcore_barrier()
            for j in range(ROWS_PER_TILE // EB):
                pltpu.sync_copy(rows0, acc_sh.at[pl.ds(base + j * EB, EB)])
            _fill_rows(rows1, EB, jnp.ones((16,), _f32))

            plsc.subcore_barrier()

            def dchunk(k, _):
                pltpu.sync_copy(dst_hbm.at[w, pl.ds(k * NBC, NBC)], dst_v)

                def batch(i, _):
                    pltpu.sync_copy(rows1, acc_sh.at[dst_v.at[i]], add=True)
                    return 0

                lax.fori_loop(0, NBC, batch, 0)
                return 0

            lax.fori_loop(0, NCHUNK, dchunk, 0)

            plsc.subcore_barrier()

            for j in range(ROWS_PER_TILE // EB):
                o = base + j * EB
                pltpu.sync_copy(acc_sh.at[pl.ds(o, EB)], rows0)
                pltpu.sync_copy(rows0, deg_hbm.at[c, pl.ds(o, EB)])

    out_type = [jax.ShapeDtypeStruct((NCORES, NPAD, D), _f32)]
    if with_deg:
        out_type.append(jax.ShapeDtypeStruct((NCORES, NPAD, D), _f32))
    return pl.kernel(body, out_type=out_type, mesh=_sc_mesh,
                     scratch_types=scratch)


_sc_scatter_deg = _make_sc_scatter(with_deg=True)
_sc_scatter = _make_sc_scatter(with_deg=False)


@jax.jit
def kernel(h, edge_index, W1_self, W1_neigh, b1, W2_self, W2_neigh, b2):
    src = edge_index[0].reshape(NW, EPW)
    dst = edge_index[1].reshape(NW, EPW)
    padlen = EPW_PAD - EPW
    pad_idx = (N + (jnp.arange(padlen, dtype=jnp.int32) % 8))[None, :]
    pad_blk = jnp.broadcast_to(pad_idx, (NW, padlen))
    src_p = jnp.concatenate([src, pad_blk], axis=1).reshape(NW, NB, EB)
    dst_p = jnp.concatenate([dst, pad_blk], axis=1).reshape(NW, NB, EB)

    b1r = b1[None, :]
    b2r = b2[None, :]

    y1 = _project(h, W1_neigh)
    agg1, deg = _sc_scatter_deg(y1, src_p, dst_p)
    x1, y2, dinv = _combine1(h, agg1, deg, W1_self, b1r, W2_neigh)
    (agg2,) = _sc_scatter(y2, src_p, dst_p)
    x2t = _combine2(x1, agg2, dinv, W2_self, b2r)
    return x2t[:, :N]
